# Initial kernel scaffold; baseline (speedup 1.0000x reference)
import jax
import jax.numpy as jnp
from jax.experimental import pallas as pl


def kernel(x, W):
    raise NotImplementedError("write your pallas kernel here")


# speedup vs baseline: 2.4872x; 2.4872x over previous
import functools

import jax
import jax.numpy as jnp
from jax import lax
from jax.experimental import pallas as pl
from jax.experimental.pallas import tpu as pltpu
from jax.experimental.pallas import tpu_sc as plsc

D = 16
G = 128
K = 16
CH = G * K


@functools.cache
def _make_gather(B):
    info = plsc.get_sparse_core_info()
    NW = info.num_cores * info.num_subcores
    NC = info.num_cores
    b_per_w = B // NW
    assert b_per_w * NW == B and b_per_w % CH == 0
    n_chunks = b_per_w // CH
    rows_per_w = b_per_w // G

    mesh = plsc.VectorSubcoreMesh(core_axis_name="c", subcore_axis_name="s")

    @functools.partial(
        pl.kernel,
        mesh=mesh,
        out_type=jax.ShapeDtypeStruct((B, D), jnp.float32),
        scratch_types=[
            pltpu.VMEM((K, G), jnp.int32),
            pltpu.VMEM((CH, D), jnp.float32),
            pltpu.SemaphoreType.DMA,
        ],
        compiler_params=pltpu.CompilerParams(use_tc_tiling_on_sc=False),
    )
    def gather_kernel(idx_hbm, table_hbm, out_hbm, idx_v, rows_v, sem):
        wid = lax.axis_index("s") * NC + lax.axis_index("c")
        row_base = wid * rows_per_w
        out_base = wid * b_per_w

        def body(c, carry):
            pltpu.sync_copy(idx_hbm.at[pl.ds(row_base + c * K, K)], idx_v)
            cps = [
                pltpu.async_copy(
                    table_hbm.at[idx_v.at[j]], rows_v.at[pl.ds(j * G, G)], sem
                )
                for j in range(K)
            ]
            for cp in cps:
                cp.wait()
            pltpu.sync_copy(rows_v, out_hbm.at[pl.ds(out_base + c * CH, CH)])
            return carry

        lax.fori_loop(0, n_chunks, body, 0)

    return gather_kernel


def kernel(x, W):
    B = x.shape[0] * x.shape[1]
    idx = x.reshape(B // G, G).astype(jnp.int32)
    out = _make_gather(B)(idx, W)
    return out.reshape(x.shape[0], x.shape[1], D)


# speedup vs baseline: 2.5686x; 1.0327x over previous
import functools

import jax
import jax.numpy as jnp
from jax import lax
from jax.experimental import pallas as pl
from jax.experimental.pallas import tpu as pltpu
from jax.experimental.pallas import tpu_sc as plsc

D = 16
G = 128
K = 20
CH = G * K


@functools.cache
def _make_gather(B):
    info = plsc.get_sparse_core_info()
    NW = info.num_cores * info.num_subcores
    NC = info.num_cores
    b_per_w = B // NW
    assert b_per_w * NW == B and b_per_w % CH == 0
    n_chunks = b_per_w // CH
    assert n_chunks % 2 == 0 and n_chunks >= 4
    rows_per_w = b_per_w // G

    mesh = plsc.VectorSubcoreMesh(core_axis_name="c", subcore_axis_name="s")

    @functools.partial(
        pl.kernel,
        mesh=mesh,
        out_type=jax.ShapeDtypeStruct((B, D), jnp.float32),
        scratch_types=[
            pltpu.VMEM((2, K, G), jnp.int32),
            pltpu.VMEM((2, CH, D), jnp.float32),
            pltpu.SemaphoreType.DMA,
            pltpu.SemaphoreType.DMA,
            pltpu.SemaphoreType.DMA,
            pltpu.SemaphoreType.DMA,
        ],
        compiler_params=pltpu.CompilerParams(use_tc_tiling_on_sc=False),
    )
    def gather_kernel(idx_hbm, table_hbm, out_hbm, idx_v, rows_v, g0, g1, s0, s1):
        wid = lax.axis_index("s") * NC + lax.axis_index("c")
        row_base = wid * rows_per_w
        out_base = wid * b_per_w
        gsem = (g0, g1)
        ssem = (s0, s1)

        def load_idx(c, b):
            pltpu.sync_copy(idx_hbm.at[pl.ds(row_base + c * K, K)], idx_v.at[b])

        def fire(c, b):
            for j in range(K):
                pltpu.async_copy(
                    table_hbm.at[idx_v.at[b, j]],
                    rows_v.at[b, pl.ds(j * G, G)],
                    gsem[b],
                )

        def drain_gathers(b):
            for j in range(K):
                pltpu.make_async_copy(
                    table_hbm.at[idx_v.at[b, j]],
                    rows_v.at[b, pl.ds(j * G, G)],
                    gsem[b],
                ).wait()

        def store(c, b):
            return pltpu.async_copy(
                rows_v.at[b], out_hbm.at[pl.ds(out_base + c * CH, CH)], ssem[b]
            )

        def wait_store(b):
            pltpu.make_async_copy(
                rows_v.at[b], out_hbm.at[pl.ds(out_base, CH)], ssem[b]
            ).wait()

        load_idx(0, 0)
        fire(0, 0)
        load_idx(1, 1)
        fire(1, 1)
        drain_gathers(0)
        store(0, 0)

        def loop_body(i, carry):
            c2 = 2 * i
            for b in range(2):
                c = c2 + b
                ob = 1 - b
                wait_store(b)
                load_idx(c, b)
                fire(c, b)
                drain_gathers(ob)
                store(c - 1, ob)
            return carry

        lax.fori_loop(1, n_chunks // 2, loop_body, 0)

        drain_gathers(1)
        wait_store(0)
        store(n_chunks - 1, 1).wait()

    return gather_kernel


def kernel(x, W):
    B = x.shape[0] * x.shape[1]
    idx = x.reshape(B // G, G).astype(jnp.int32)
    out = _make_gather(B)(idx, W)
    return out.reshape(x.shape[0], x.shape[1], D)


# speedup vs baseline: 4.7240x; 1.8392x over previous
import functools

import jax
import jax.numpy as jnp
from jax import lax
from jax.experimental import pallas as pl
from jax.experimental.pallas import tpu as pltpu
from jax.experimental.pallas import tpu_sc as plsc

D = 16
HB = 8
LB = 128
NH = 200
NB = 16384
HT = NH // HB
CT = NB // LB
N_SB = HT * CT


@functools.cache
def _make_gather():
    info = plsc.get_sparse_core_info()
    NW = info.num_cores * info.num_subcores
    NC = info.num_cores
    sb_per_w = N_SB // NW
    assert sb_per_w % 2 == 0 and sb_per_w >= 6

    mesh = plsc.VectorSubcoreMesh(core_axis_name="c", subcore_axis_name="s")

    @functools.partial(
        pl.kernel,
        mesh=mesh,
        out_type=jax.ShapeDtypeStruct((NH, 2, CT, 8, LB), jnp.float32),
        scratch_types=[
            pltpu.VMEM((2, HB, LB), jnp.int32),
            pltpu.VMEM((2, HB, LB, D), jnp.float32),
            pltpu.VMEM((2, 2, HB, 8, LB), jnp.float32),
            pltpu.SemaphoreType.DMA,
            pltpu.SemaphoreType.DMA,
            pltpu.SemaphoreType.DMA,
            pltpu.SemaphoreType.DMA,
        ],
        compiler_params=pltpu.CompilerParams(
            use_tc_tiling_on_sc=False, needs_layout_passes=False
        ),
    )
    def gather_kernel(xn, table, out5, idx_v, rows_v, obuf, g0, g1, s0, s1):
        wid = lax.axis_index("s") * NC + lax.axis_index("c")
        sb0 = wid * sb_per_w
        gsem = (g0, g1)
        ssem = (s0, s1)
        iota = lax.iota(jnp.int32, 16)

        def coords(sb):
            return sb // CT, sb % CT

        def load_and_fire(sb, b):
            ht, ct = coords(sb)
            pltpu.sync_copy(xn.at[ht, ct], idx_v.at[b])
            for hi in range(HB):
                pltpu.async_copy(
                    table.at[idx_v.at[b, hi]], rows_v.at[b, hi], gsem[b]
                )

        def drain_gathers(b):
            for hi in range(HB):
                pltpu.make_async_copy(
                    table.at[idx_v.at[b, hi]], rows_v.at[b, hi], gsem[b]
                ).wait()

        def transpose(b):
            b_v = jnp.full((16,), b, jnp.int32)

            def tbody(t, carry):
                hi = t // 2
                dt = t % 2
                hi_v = jnp.zeros((16,), jnp.int32) + hi
                for din in range(8):
                    d_v = jnp.full((16,), din, jnp.int32) + dt * 8
                    for l0 in range(0, LB, 16):
                        v = plsc.load_gather(
                            rows_v, [b_v, hi_v, iota + l0, d_v]
                        )
                        obuf[b, dt, hi, din, pl.ds(l0, 16)] = v
                return carry

            lax.fori_loop(0, HB * 2, tbody, 0)

        def store(sb, b):
            ht, ct = coords(sb)
            for dt in range(2):
                pltpu.async_copy(
                    obuf.at[b, dt],
                    out5.at[pl.ds(ht * HB, HB), dt, ct],
                    ssem[b],
                )

        def wait_store(b):
            for dt in range(2):
                pltpu.make_async_copy(
                    obuf.at[b, dt], out5.at[pl.ds(0, HB), dt, 0], ssem[b]
                ).wait()

        load_and_fire(sb0 + 0, 0)
        load_and_fire(sb0 + 1, 1)
        drain_gathers(0)
        transpose(0)
        store(sb0 + 0, 0)
        load_and_fire(sb0 + 2, 0)
        drain_gathers(1)
        transpose(1)
        store(sb0 + 1, 1)
        load_and_fire(sb0 + 3, 1)

        def loop_body(i, carry):
            for b in range(2):
                c = 2 * i + b
                drain_gathers(b)
                wait_store(b)
                transpose(b)
                store(sb0 + c, b)
                load_and_fire(sb0 + c + 2, b)
            return carry

        lax.fori_loop(1, sb_per_w // 2 - 1, loop_body, 0)

        for b in range(2):
            c = sb_per_w - 2 + b
            drain_gathers(b)
            wait_store(b)
            transpose(b)
            store(sb0 + c, b)
        wait_store(0)
        wait_store(1)

    return gather_kernel


def kernel(x, W):
    xn = (
        x.T.astype(jnp.int32)
        .reshape(HT, HB, CT, LB)
        .transpose(0, 2, 1, 3)
    )
    o5 = _make_gather()(xn, W)
    return o5.transpose(2, 4, 0, 1, 3).reshape(NB, NH, D)
